# Initial kernel scaffold; baseline (speedup 1.0000x reference)
import jax
import jax.numpy as jnp
from jax.experimental import pallas as pl


def kernel(inputs):
    raise NotImplementedError("write your pallas kernel here")


# speedup vs baseline: 5.5541x; 5.5541x over previous
import functools

import jax
import jax.numpy as jnp
from jax import lax
from jax.experimental import pallas as pl
from jax.experimental.pallas import tpu as pltpu
from jax.experimental.pallas import tpu_sc as plsc

BATCH = 16384
ROW = 2048
NSEG = 128
NBLK = ROW // NSEG
LANES = 16
VPB = NSEG // LANES

_info = plsc.get_sparse_core_info()
_NC, _NS = _info.num_cores, _info.num_subcores
NW = _NC * _NS
ROWS_PER_W = BATCH // NW
CHUNK = 8
NCHUNK = ROWS_PER_W // CHUNK


def _sort128_desc(keys, vals):
    def merge_bitonic(ks, vs):
        m = len(ks)
        if m == 1:
            k, v = plsc.sort_key_val(ks[0], vs[0], descending=True)
            return [k], [v]
        h = m // 2
        hi_k, hi_v, lo_k, lo_v = [], [], [], []
        for i in range(h):
            c = ks[i] >= ks[i + h]
            hi_k.append(jnp.where(c, ks[i], ks[i + h]))
            lo_k.append(jnp.where(c, ks[i + h], ks[i]))
            hi_v.append(jnp.where(c, vs[i], vs[i + h]))
            lo_v.append(jnp.where(c, vs[i + h], vs[i]))
        rk1, rv1 = merge_bitonic(hi_k, hi_v)
        rk2, rv2 = merge_bitonic(lo_k, lo_v)
        return rk1 + rk2, rv1 + rv2

    def merge_two(ka, va, kb, vb):
        ks = ka + [lax.rev(x, (0,)) for x in reversed(kb)]
        vs = va + [lax.rev(x, (0,)) for x in reversed(vb)]
        return merge_bitonic(ks, vs)

    runs = []
    for k, v in zip(keys, vals):
        sk, sv = plsc.sort_key_val(k, v, descending=True)
        runs.append(([sk], [sv]))
    while len(runs) > 1:
        nxt = []
        for i in range(0, len(runs), 2):
            (ka, va), (kb, vb) = runs[i], runs[i + 1]
            nxt.append(merge_two(ka, va, kb, vb))
        runs = nxt
    return runs[0]


_mesh = plsc.VectorSubcoreMesh(core_axis_name="c", subcore_axis_name="s")


@functools.partial(
    pl.kernel,
    mesh=_mesh,
    out_type=jax.ShapeDtypeStruct((BATCH, ROW), jnp.float32),
    scratch_types=[
        pltpu.VMEM((CHUNK, ROW), jnp.float32),
        pltpu.VMEM((CHUNK, ROW), jnp.float32),
    ],
    compiler_params=pltpu.CompilerParams(needs_layout_passes=False),
)
def _perm_canon(in_hbm, out_hbm, in_buf, out_buf):
    wid = lax.axis_index("s") * _NC + lax.axis_index("c")
    base_row = wid * ROWS_PER_W
    iota = lax.iota(jnp.int32, LANES)

    def row_body(r, carry):
        keys, vals = [], []
        for v in range(VPB):
            re = in_buf[r, pl.ds(v * LANES, LANES)]
            im = in_buf[r, pl.ds(ROW // 2 + v * LANES, LANES)]
            keys.append(re * re + im * im)
            vals.append(iota + (v * LANES))
        _, sv = _sort128_desc(keys, vals)
        r_vec = jnp.zeros((LANES,), jnp.int32) + r
        for b in range(NBLK):
            for v in range(VPB):
                idxv = sv[v] + (b * NSEG)
                g = plsc.load_gather(in_buf, [r_vec, idxv])
                out_buf[r, pl.ds(b * NSEG + v * LANES, LANES)] = g
        return carry

    def chunk_body(ci, carry):
        row0 = base_row + ci * CHUNK
        pltpu.sync_copy(in_hbm.at[pl.ds(row0, CHUNK)], in_buf)
        lax.fori_loop(0, CHUNK, row_body, 0)
        pltpu.sync_copy(out_buf, out_hbm.at[pl.ds(row0, CHUNK)])
        return carry

    lax.fori_loop(0, NCHUNK, chunk_body, 0)


def kernel(inputs):
    return _perm_canon(inputs)


# speedup vs baseline: 10.3180x; 1.8577x over previous
import functools

import jax
import jax.numpy as jnp
from jax import lax
from jax.experimental import pallas as pl
from jax.experimental.pallas import tpu as pltpu
from jax.experimental.pallas import tpu_sc as plsc

BATCH = 16384
ROW = 2048
NSEG = 128
NBLK = ROW // NSEG
LANES = 16
VPB = NSEG // LANES

_info = plsc.get_sparse_core_info()
_NC, _NS = _info.num_cores, _info.num_subcores
NW = _NC * _NS
ROWS_PER_W = BATCH // NW
CHUNK = 8
NCHUNK = ROWS_PER_W // CHUNK


def _sort128_desc(keys, vals):
    def merge_bitonic(ks, vs):
        m = len(ks)
        if m == 1:
            k, v = plsc.sort_key_val(ks[0], vs[0], descending=True)
            return [k], [v]
        h = m // 2
        hi_k, hi_v, lo_k, lo_v = [], [], [], []
        for i in range(h):
            c = ks[i] >= ks[i + h]
            hi_k.append(jnp.where(c, ks[i], ks[i + h]))
            lo_k.append(jnp.where(c, ks[i + h], ks[i]))
            hi_v.append(jnp.where(c, vs[i], vs[i + h]))
            lo_v.append(jnp.where(c, vs[i + h], vs[i]))
        rk1, rv1 = merge_bitonic(hi_k, hi_v)
        rk2, rv2 = merge_bitonic(lo_k, lo_v)
        return rk1 + rk2, rv1 + rv2

    def merge_two(ka, va, kb, vb):
        ks = ka + [lax.rev(x, (0,)) for x in reversed(kb)]
        vs = va + [lax.rev(x, (0,)) for x in reversed(vb)]
        return merge_bitonic(ks, vs)

    runs = []
    for k, v in zip(keys, vals):
        sk, sv = plsc.sort_key_val(k, v, descending=True)
        runs.append(([sk], [sv]))
    while len(runs) > 1:
        nxt = []
        for i in range(0, len(runs), 2):
            (ka, va), (kb, vb) = runs[i], runs[i + 1]
            nxt.append(merge_two(ka, va, kb, vb))
        runs = nxt
    return runs[0]


_mesh = plsc.VectorSubcoreMesh(core_axis_name="c", subcore_axis_name="s")


@functools.partial(
    pl.kernel,
    mesh=_mesh,
    out_type=jax.ShapeDtypeStruct((BATCH, ROW), jnp.float32),
    scratch_types=[
        pltpu.VMEM((2 * CHUNK, ROW), jnp.float32),
        pltpu.VMEM((2 * CHUNK, ROW), jnp.float32),
        pltpu.SemaphoreType.DMA((2,)),
        pltpu.SemaphoreType.DMA((2,)),
    ],
    compiler_params=pltpu.CompilerParams(needs_layout_passes=False),
)
def _perm_canon(in_hbm, out_hbm, in_buf, out_buf, sem_in, sem_out):
    wid = lax.axis_index("s") * _NC + lax.axis_index("c")
    base_row = wid * ROWS_PER_W
    iota = lax.iota(jnp.int32, LANES)

    def load(ci, slot):
        row0 = base_row + ci * CHUNK
        return pltpu.make_async_copy(
            in_hbm.at[pl.ds(row0, CHUNK)],
            in_buf.at[pl.ds(slot * CHUNK, CHUNK)], sem_in.at[slot])

    def store(ci, slot):
        row0 = base_row + ci * CHUNK
        return pltpu.make_async_copy(
            out_buf.at[pl.ds(slot * CHUNK, CHUNK)],
            out_hbm.at[pl.ds(row0, CHUNK)], sem_out.at[slot])

    load(0, 0).start()

    def chunk_body(ci, carry):
        slot = lax.rem(ci, 2)

        @pl.when(ci + 1 < NCHUNK)
        def _prefetch():
            load(ci + 1, 1 - slot).start()

        load(ci, slot).wait()

        @pl.when(ci >= 2)
        def _drain():
            store(ci - 2, slot).wait()

        rbase = slot * CHUNK

        @plsc.parallel_loop(0, CHUNK, unroll=2)
        def row_body(r):
            rr = rbase + r
            keys, vals = [], []
            for v in range(VPB):
                re = in_buf[rr, pl.ds(v * LANES, LANES)]
                im = in_buf[rr, pl.ds(ROW // 2 + v * LANES, LANES)]
                keys.append(re * re + im * im)
                vals.append(iota + (v * LANES))
            _, sv = _sort128_desc(keys, vals)
            r_vec = jnp.zeros((LANES,), jnp.int32) + rr
            for b in range(NBLK):
                for v in range(VPB):
                    idxv = sv[v] + (b * NSEG)
                    g = plsc.load_gather(in_buf, [r_vec, idxv])
                    out_buf[rr, pl.ds(b * NSEG + v * LANES, LANES)] = g

        store(ci, slot).start()
        return carry

    lax.fori_loop(0, NCHUNK, chunk_body, 0)
    store(NCHUNK - 2, lax.rem(NCHUNK - 2, 2)).wait()
    store(NCHUNK - 1, lax.rem(NCHUNK - 1, 2)).wait()


def kernel(inputs):
    return _perm_canon(inputs)
